# baseline (speedup 1.0000x reference)
import functools

import jax
import jax.numpy as jnp
from jax import lax
from jax.experimental import pallas as pl
from jax.experimental.pallas import tpu as pltpu
from jax.experimental.pallas import tpu_sc as plsc

_SCALING = 2.5
_LANES = 128


def _pack_rows(a):
    n, d = a.shape
    return lax.bitcast_convert_type(a.reshape(n, d // 2, 2), jnp.int32)


def _unpack_rows(a):
    n, k = a.shape
    return lax.bitcast_convert_type(a, jnp.bfloat16).reshape(n, 2 * k)


def _sc_gather(table, idx, chunk=16):
    n, = idx.shape
    _, d = table.shape
    info = plsc.get_sparse_core_info()
    nc, ns = info.num_cores, info.num_subcores
    nw = nc * ns
    assert n % (nw * chunk) == 0, (n, nw, chunk)
    bpw = n // nw
    nch = bpw // chunk
    mesh = plsc.VectorSubcoreMesh(core_axis_name="c", subcore_axis_name="s")

    @functools.partial(
        pl.kernel, mesh=mesh,
        out_type=jax.ShapeDtypeStruct((n, d), table.dtype),
        scratch_types=[
            pltpu.VMEM((bpw,), jnp.int32),
            pltpu.VMEM((2, chunk, d), table.dtype),
            pltpu.SemaphoreType.DMA,
            pltpu.SemaphoreType.DMA,
        ],
    )
    def k(table_hbm, idx_hbm, out_hbm, idx_v, rows_v, sem0, sem1):
        wid = lax.axis_index("s") * nc + lax.axis_index("c")
        base = wid * bpw
        pltpu.sync_copy(idx_hbm.at[pl.ds(base, bpw)], idx_v)
        sems = (sem0, sem1)
        cps = [None, None]
        cps[0] = pltpu.async_copy(
            table_hbm.at[idx_v.at[pl.ds(0, chunk)]], rows_v.at[0], sems[0])
        for c in range(nch):
            cur = c % 2
            nxt = (c + 1) % 2
            if c + 1 < nch:
                cps[nxt] = pltpu.async_copy(
                    table_hbm.at[idx_v.at[pl.ds((c + 1) * chunk, chunk)]],
                    rows_v.at[nxt], sems[nxt])
            cps[cur].wait()
            pltpu.sync_copy(rows_v.at[cur], out_hbm.at[pl.ds(base + c * chunk, chunk)])

    return k(table, idx)


def _router_body(x_ref, gwt_ref, bias_ref, idx_ref, w_ref):
    logits = jnp.dot(x_ref[...], gwt_ref[...], preferred_element_type=jnp.float32)
    s = jax.nn.sigmoid(logits) + bias_ref[...]
    lane = lax.broadcasted_iota(jnp.int32, s.shape, 1)
    big = jnp.int32(2 ** 30)
    v1 = jnp.max(s, axis=1, keepdims=True)
    i1 = jnp.min(jnp.where(s == v1, lane, big), axis=1, keepdims=True)
    s2 = jnp.where(lane == i1, jnp.float32(-1e30), s)
    v2 = jnp.max(s2, axis=1, keepdims=True)
    i2 = jnp.min(jnp.where(s2 == v2, lane, big), axis=1, keepdims=True)
    denom = v1 + v2 + 1e-20
    w1 = v1 / denom * _SCALING
    w2 = v2 / denom * _SCALING
    idx_ref[...] = jnp.where(lane == 0, i1, jnp.where(lane == 1, i2, 0)).astype(jnp.int32)
    w_ref[...] = jnp.where(lane == 0, w1, jnp.where(lane == 1, w2, 0.0))


def _route(x_flat, gate_weight, gate_bias):
    t, h = x_flat.shape
    e = gate_weight.shape[0]
    tb = min(256, t)
    gwt = jnp.zeros((h, _LANES), jnp.float32).at[:, :e].set(gate_weight.T)
    bias = jnp.full((1, _LANES), -1e30, jnp.float32).at[0, :e].set(gate_bias)
    idx, w = pl.pallas_call(
        _router_body,
        grid=(t // tb,),
        in_specs=[
            pl.BlockSpec((tb, h), lambda i: (i, 0)),
            pl.BlockSpec((h, _LANES), lambda i: (0, 0)),
            pl.BlockSpec((1, _LANES), lambda i: (0, 0)),
        ],
        out_specs=[
            pl.BlockSpec((tb, _LANES), lambda i: (i, 0)),
            pl.BlockSpec((tb, _LANES), lambda i: (i, 0)),
        ],
        out_shape=[
            jax.ShapeDtypeStruct((t, _LANES), jnp.int32),
            jax.ShapeDtypeStruct((t, _LANES), jnp.float32),
        ],
    )(x_flat, gwt, bias)
    return idx[:, 0], idx[:, 1], w[:, 0], w[:, 1]


def _mlp_body(te_ref, tv_ref, xs_ref, wv_ref, wg_ref, w2_ref, bv_ref, bg_ref,
              b2_ref, rw_ref, out_ref, acc_ref):
    i = pl.program_id(0)
    j = pl.program_id(1)
    nj = pl.num_programs(1)

    @pl.when(tv_ref[i] == 1)
    def _():
        x = xs_ref[...]
        dn = (((1,), (1,)), ((), ()))
        up_v = lax.dot_general(x, wv_ref[0], dn, preferred_element_type=jnp.float32) + bv_ref[0]
        up_g = lax.dot_general(x, wg_ref[0], dn, preferred_element_type=jnp.float32) + bg_ref[0]
        hid = (up_v * jax.nn.sigmoid(up_v) * up_g).astype(jnp.bfloat16)
        part = lax.dot_general(hid, w2_ref[0], dn, preferred_element_type=jnp.float32)

        @pl.when(j == 0)
        def _():
            acc_ref[...] = part

        @pl.when(j > 0)
        def _():
            acc_ref[...] += part

        @pl.when(j == nj - 1)
        def _():
            out_ref[...] = ((acc_ref[...] + b2_ref[0]) * rw_ref[:, :1]).astype(jnp.bfloat16)


def _grouped_mlp(x_sorted, row_w, tile_expert, tile_valid, fc1_w, fc1_b, fc2_w, fc2_b,
                 tile):
    r, h = x_sorted.shape
    e, two_i, _ = fc1_w.shape
    inter = two_i // 2
    ip = -(-inter // 512) * 512
    ib = 768 if ip % 768 == 0 else (512 if ip % 512 == 0 else ip)
    nj = ip // ib
    pad = ip - inter
    max_tiles = r // tile
    wv = jnp.pad(fc1_w[:, :inter, :], ((0, 0), (0, pad), (0, 0))).astype(jnp.bfloat16)
    wg = jnp.pad(fc1_w[:, inter:, :], ((0, 0), (0, pad), (0, 0))).astype(jnp.bfloat16)
    w2b = jnp.pad(fc2_w, ((0, 0), (0, 0), (0, pad))).astype(jnp.bfloat16)
    bv = jnp.pad(fc1_b[:, :inter], ((0, 0), (0, pad))).reshape(e, 1, ip)
    bg = jnp.pad(fc1_b[:, inter:], ((0, 0), (0, pad))).reshape(e, 1, ip)
    b2 = fc2_b.reshape(e, 1, h)
    rw2d = jnp.broadcast_to(row_w[:, None], (r, _LANES))

    grid_spec = pltpu.PrefetchScalarGridSpec(
        num_scalar_prefetch=2,
        grid=(max_tiles, nj),
        in_specs=[
            pl.BlockSpec((tile, h), lambda i, j, te, tv: (i, 0)),
            pl.BlockSpec((1, ib, h), lambda i, j, te, tv: (te[i], j, 0)),
            pl.BlockSpec((1, ib, h), lambda i, j, te, tv: (te[i], j, 0)),
            pl.BlockSpec((1, h, ib), lambda i, j, te, tv: (te[i], 0, j)),
            pl.BlockSpec((1, 1, ib), lambda i, j, te, tv: (te[i], 0, j)),
            pl.BlockSpec((1, 1, ib), lambda i, j, te, tv: (te[i], 0, j)),
            pl.BlockSpec((1, 1, h), lambda i, j, te, tv: (te[i], 0, 0)),
            pl.BlockSpec((tile, _LANES), lambda i, j, te, tv: (i, 0)),
        ],
        out_specs=pl.BlockSpec((tile, h), lambda i, j, te, tv: (i, 0)),
        scratch_shapes=[pltpu.VMEM((tile, h), jnp.float32)],
    )
    return pl.pallas_call(
        _mlp_body,
        grid_spec=grid_spec,
        out_shape=jax.ShapeDtypeStruct((r, h), jnp.bfloat16),
    )(tile_expert, tile_valid, x_sorted, wv, wg, w2b, bv, bg, b2, rw2d)


def _shared_body(x_ref, sv_ref, sg_ref, s2_ref, bv_ref, bg_ref, b2_ref,
                 out_ref, acc_ref):
    j = pl.program_id(1)
    nj = pl.num_programs(1)
    x = x_ref[...].astype(jnp.bfloat16)
    dn = (((1,), (1,)), ((), ()))
    up_v = lax.dot_general(x, sv_ref[...], dn, preferred_element_type=jnp.float32) + bv_ref[...]
    up_g = lax.dot_general(x, sg_ref[...], dn, preferred_element_type=jnp.float32) + bg_ref[...]
    hid = (up_v * jax.nn.sigmoid(up_v) * up_g).astype(jnp.bfloat16)
    part = lax.dot_general(hid, s2_ref[...], dn, preferred_element_type=jnp.float32)

    @pl.when(j == 0)
    def _():
        acc_ref[...] = part

    @pl.when(j > 0)
    def _():
        acc_ref[...] += part

    @pl.when(j == nj - 1)
    def _():
        out_ref[...] = acc_ref[...] + b2_ref[...]


def _shared_mlp(x_flat, sh_fc1_w, sh_fc1_b, sh_fc2_w, sh_fc2_b, tile):
    t, h = x_flat.shape
    nt = t // tile
    sh_inter = sh_fc2_w.shape[1]
    sp = -(-sh_inter // 768) * 768
    sib = 768
    nj = sp // sib
    pad = sp - sh_inter
    sv = jnp.pad(sh_fc1_w[:sh_inter, :], ((0, pad), (0, 0))).astype(jnp.bfloat16)
    sg = jnp.pad(sh_fc1_w[sh_inter:, :], ((0, pad), (0, 0))).astype(jnp.bfloat16)
    s2b = jnp.pad(sh_fc2_w, ((0, 0), (0, pad))).astype(jnp.bfloat16)
    bv = jnp.pad(sh_fc1_b[:sh_inter], (0, pad)).reshape(1, sp)
    bg = jnp.pad(sh_fc1_b[sh_inter:], (0, pad)).reshape(1, sp)
    b2 = sh_fc2_b.reshape(1, h)
    return pl.pallas_call(
        _shared_body,
        grid=(nt, nj),
        in_specs=[
            pl.BlockSpec((tile, h), lambda i, j: (i, 0)),
            pl.BlockSpec((sib, h), lambda i, j: (j, 0)),
            pl.BlockSpec((sib, h), lambda i, j: (j, 0)),
            pl.BlockSpec((h, sib), lambda i, j: (0, j)),
            pl.BlockSpec((1, sib), lambda i, j: (0, j)),
            pl.BlockSpec((1, sib), lambda i, j: (0, j)),
            pl.BlockSpec((1, h), lambda i, j: (0, 0)),
        ],
        out_specs=pl.BlockSpec((tile, h), lambda i, j: (i, 0)),
        out_shape=jax.ShapeDtypeStruct((t, h), jnp.float32),
        scratch_shapes=[pltpu.VMEM((tile, h), jnp.float32)],
    )(x_flat, sv, sg, s2b, bv, bg, b2)


def _combine_body(sh_ref, g0_ref, g1_ref, out_ref):
    out_ref[...] = (sh_ref[...] + g0_ref[...].astype(jnp.float32)
                    + g1_ref[...].astype(jnp.float32))


def _combine(shared_out, gcat, tile):
    t, h = shared_out.shape
    nt = t // tile
    return pl.pallas_call(
        _combine_body,
        grid=(nt,),
        in_specs=[
            pl.BlockSpec((tile, h), lambda i: (i, 0)),
            pl.BlockSpec((tile, h), lambda i: (i, 0)),
            pl.BlockSpec((tile, h), lambda i: (i + nt, 0)),
        ],
        out_specs=pl.BlockSpec((tile, h), lambda i: (i, 0)),
        out_shape=jax.ShapeDtypeStruct((t, h), jnp.float32),
    )(shared_out, gcat, gcat)


def kernel(x, gate_weight, gate_bias, fc1_w, fc1_b, fc2_w, fc2_b,
           sh_fc1_w, sh_fc1_b, sh_fc2_w, sh_fc2_b):
    b, s, h = x.shape
    t = b * s
    e = gate_weight.shape[0]
    x_flat = x.reshape(t, h)
    tile = min(256, t)
    p = 2 * t
    max_tiles = p // tile + e
    r = max_tiles * tile

    i1, i2, w1, w2 = _route(x_flat, gate_weight, gate_bias)

    e_all = jnp.concatenate([i1, i2])
    w_all = jnp.concatenate([w1, w2])
    perm = jnp.argsort(e_all, stable=True)
    e_sorted = e_all[perm]
    tok_sorted = (perm % t).astype(jnp.int32)
    counts = jnp.bincount(e_all, length=e)
    tiles_per_e = (counts + tile - 1) // tile
    seg_start = (jnp.concatenate([jnp.zeros((1,), jnp.int32),
                                  jnp.cumsum(tiles_per_e)[:-1].astype(jnp.int32)]) * tile)
    orig_start = jnp.concatenate([jnp.zeros((1,), jnp.int32),
                                  jnp.cumsum(counts)[:-1].astype(jnp.int32)])
    rank = jnp.arange(p, dtype=jnp.int32) - orig_start[e_sorted]
    dst = seg_start[e_sorted] + rank
    row_tok = jnp.zeros((r,), jnp.int32).at[dst].set(tok_sorted)
    row_w = jnp.zeros((r,), jnp.float32).at[dst].set(w_all[perm])
    inv = jnp.zeros((p,), jnp.int32).at[perm].set(dst)
    num_tiles_used = jnp.sum(tiles_per_e).astype(jnp.int32)
    tile_expert = jnp.repeat(jnp.arange(e, dtype=jnp.int32), tiles_per_e,
                             total_repeat_length=max_tiles)
    tile_expert = jnp.clip(tile_expert, 0, e - 1).astype(jnp.int32)
    tile_valid = (jnp.arange(max_tiles, dtype=jnp.int32) < num_tiles_used).astype(jnp.int32)

    shared_out = _shared_mlp(x_flat, sh_fc1_w, sh_fc1_b, sh_fc2_w, sh_fc2_b, tile)

    x_sorted = _unpack_rows(_sc_gather(_pack_rows(x_flat.astype(jnp.bfloat16)),
                                       row_tok))

    y_sorted = _grouped_mlp(x_sorted, row_w, tile_expert, tile_valid,
                            fc1_w, fc1_b, fc2_w, fc2_b, tile)

    gcat = _unpack_rows(_sc_gather(_pack_rows(y_sorted), inv))

    out = _combine(shared_out, gcat, tile)
    return out.reshape(b, s, h)


# speedup vs baseline: 1.7973x; 1.7973x over previous
import functools

import jax
import jax.numpy as jnp
from jax import lax
from jax.experimental import pallas as pl
from jax.experimental.pallas import tpu as pltpu
from jax.experimental.pallas import tpu_sc as plsc

_SCALING = 2.5
_LANES = 128


def _sc_gather(table, idx):
    n, = idx.shape
    _, d = table.shape
    info = plsc.get_sparse_core_info()
    nc, ns = info.num_cores, info.num_subcores
    nw = nc * ns
    bpw = n // nw
    budget = 480 * 1024 // (2 * d * table.dtype.itemsize)
    chunk = next(c for c in (32, 24, 16, 8) if c <= budget and bpw % c == 0)
    assert n % nw == 0, (n, nw)
    nch = bpw // chunk
    mesh = plsc.VectorSubcoreMesh(core_axis_name="c", subcore_axis_name="s")

    @functools.partial(
        pl.kernel, mesh=mesh,
        out_type=jax.ShapeDtypeStruct((n, d), table.dtype),
        scratch_types=[
            pltpu.VMEM((bpw,), jnp.int32),
            pltpu.VMEM((2, chunk, d), table.dtype),
            pltpu.SemaphoreType.DMA,
            pltpu.SemaphoreType.DMA,
        ],
    )
    def k(table_hbm, idx_hbm, out_hbm, idx_v, rows_v, sem0, sem1):
        wid = lax.axis_index("s") * nc + lax.axis_index("c")
        base = wid * bpw
        pltpu.sync_copy(idx_hbm.at[pl.ds(base, bpw)], idx_v)
        sems = (sem0, sem1)
        cps = [None, None]
        cps[0] = pltpu.async_copy(
            table_hbm.at[idx_v.at[pl.ds(0, chunk)]], rows_v.at[0], sems[0])
        for c in range(nch):
            cur = c % 2
            nxt = (c + 1) % 2
            if c + 1 < nch:
                cps[nxt] = pltpu.async_copy(
                    table_hbm.at[idx_v.at[pl.ds((c + 1) * chunk, chunk)]],
                    rows_v.at[nxt], sems[nxt])
            cps[cur].wait()
            pltpu.sync_copy(rows_v.at[cur], out_hbm.at[pl.ds(base + c * chunk, chunk)])

    return k(table, idx)


def _router_body(x_ref, gwt_ref, bias_ref, idx_ref, w_ref):
    logits = jnp.dot(x_ref[...], gwt_ref[...], preferred_element_type=jnp.float32)
    s = jax.nn.sigmoid(logits) + bias_ref[...]
    lane = lax.broadcasted_iota(jnp.int32, s.shape, 1)
    big = jnp.int32(2 ** 30)
    v1 = jnp.max(s, axis=1, keepdims=True)
    i1 = jnp.min(jnp.where(s == v1, lane, big), axis=1, keepdims=True)
    s2 = jnp.where(lane == i1, jnp.float32(-1e30), s)
    v2 = jnp.max(s2, axis=1, keepdims=True)
    i2 = jnp.min(jnp.where(s2 == v2, lane, big), axis=1, keepdims=True)
    denom = v1 + v2 + 1e-20
    w1 = v1 / denom * _SCALING
    w2 = v2 / denom * _SCALING
    idx_ref[...] = jnp.where(lane == 0, i1, jnp.where(lane == 1, i2, 0)).astype(jnp.int32)
    w_ref[...] = jnp.where(lane == 0, w1, jnp.where(lane == 1, w2, 0.0))


def _route(x_flat, gate_weight, gate_bias):
    t, h = x_flat.shape
    e = gate_weight.shape[0]
    tb = min(256, t)
    gwt = jnp.zeros((h, _LANES), jnp.float32).at[:, :e].set(gate_weight.T)
    bias = jnp.full((1, _LANES), -1e30, jnp.float32).at[0, :e].set(gate_bias)
    idx, w = pl.pallas_call(
        _router_body,
        grid=(t // tb,),
        in_specs=[
            pl.BlockSpec((tb, h), lambda i: (i, 0)),
            pl.BlockSpec((h, _LANES), lambda i: (0, 0)),
            pl.BlockSpec((1, _LANES), lambda i: (0, 0)),
        ],
        out_specs=[
            pl.BlockSpec((tb, _LANES), lambda i: (i, 0)),
            pl.BlockSpec((tb, _LANES), lambda i: (i, 0)),
        ],
        out_shape=[
            jax.ShapeDtypeStruct((t, _LANES), jnp.int32),
            jax.ShapeDtypeStruct((t, _LANES), jnp.float32),
        ],
    )(x_flat, gwt, bias)
    return idx[:, 0], idx[:, 1], w[:, 0], w[:, 1]


def _mlp_body(te_ref, tv_ref, xs_ref, wv_ref, wg_ref, w2_ref, bv_ref, bg_ref,
              b2_ref, rw_ref, out_ref, acc_ref):
    i = pl.program_id(0)
    j = pl.program_id(1)
    nj = pl.num_programs(1)

    @pl.when(tv_ref[i] == 1)
    def _():
        x = xs_ref[...].astype(jnp.bfloat16)
        dn = (((1,), (1,)), ((), ()))
        up_v = lax.dot_general(x, wv_ref[0], dn, preferred_element_type=jnp.float32) + bv_ref[0]
        up_g = lax.dot_general(x, wg_ref[0], dn, preferred_element_type=jnp.float32) + bg_ref[0]
        hid = (up_v * jax.nn.sigmoid(up_v) * up_g).astype(jnp.bfloat16)
        part = lax.dot_general(hid, w2_ref[0], dn, preferred_element_type=jnp.float32)

        @pl.when(j == 0)
        def _():
            acc_ref[...] = part

        @pl.when(j > 0)
        def _():
            acc_ref[...] += part

        @pl.when(j == nj - 1)
        def _():
            out_ref[...] = (acc_ref[...] + b2_ref[0]) * rw_ref[:, :1]


def _grouped_mlp(x_sorted, row_w, tile_expert, tile_valid, fc1_w, fc1_b, fc2_w, fc2_b,
                 tile):
    r, h = x_sorted.shape
    e, two_i, _ = fc1_w.shape
    inter = two_i // 2
    ip = -(-inter // 512) * 512
    ib = 768 if ip % 768 == 0 else (512 if ip % 512 == 0 else ip)
    nj = ip // ib
    pad = ip - inter
    max_tiles = r // tile
    wv = jnp.pad(fc1_w[:, :inter, :], ((0, 0), (0, pad), (0, 0))).astype(jnp.bfloat16)
    wg = jnp.pad(fc1_w[:, inter:, :], ((0, 0), (0, pad), (0, 0))).astype(jnp.bfloat16)
    w2b = jnp.pad(fc2_w, ((0, 0), (0, 0), (0, pad))).astype(jnp.bfloat16)
    bv = jnp.pad(fc1_b[:, :inter], ((0, 0), (0, pad))).reshape(e, 1, ip)
    bg = jnp.pad(fc1_b[:, inter:], ((0, 0), (0, pad))).reshape(e, 1, ip)
    b2 = fc2_b.reshape(e, 1, h)
    rw2d = jnp.broadcast_to(row_w[:, None], (r, _LANES))

    grid_spec = pltpu.PrefetchScalarGridSpec(
        num_scalar_prefetch=2,
        grid=(max_tiles, nj),
        in_specs=[
            pl.BlockSpec((tile, h), lambda i, j, te, tv: (i, 0)),
            pl.BlockSpec((1, ib, h), lambda i, j, te, tv: (te[i], j, 0)),
            pl.BlockSpec((1, ib, h), lambda i, j, te, tv: (te[i], j, 0)),
            pl.BlockSpec((1, h, ib), lambda i, j, te, tv: (te[i], 0, j)),
            pl.BlockSpec((1, 1, ib), lambda i, j, te, tv: (te[i], 0, j)),
            pl.BlockSpec((1, 1, ib), lambda i, j, te, tv: (te[i], 0, j)),
            pl.BlockSpec((1, 1, h), lambda i, j, te, tv: (te[i], 0, 0)),
            pl.BlockSpec((tile, _LANES), lambda i, j, te, tv: (i, 0)),
        ],
        out_specs=pl.BlockSpec((tile, h), lambda i, j, te, tv: (i, 0)),
        scratch_shapes=[pltpu.VMEM((tile, h), jnp.float32)],
    )
    return pl.pallas_call(
        _mlp_body,
        grid_spec=grid_spec,
        out_shape=jax.ShapeDtypeStruct((r, h), jnp.float32),
    )(tile_expert, tile_valid, x_sorted, wv, wg, w2b, bv, bg, b2, rw2d)


def _shared_body(x_ref, sv_ref, sg_ref, s2_ref, bv_ref, bg_ref, b2_ref,
                 out_ref, acc_ref):
    j = pl.program_id(1)
    nj = pl.num_programs(1)
    x = x_ref[...].astype(jnp.bfloat16)
    dn = (((1,), (1,)), ((), ()))
    up_v = lax.dot_general(x, sv_ref[...], dn, preferred_element_type=jnp.float32) + bv_ref[...]
    up_g = lax.dot_general(x, sg_ref[...], dn, preferred_element_type=jnp.float32) + bg_ref[...]
    hid = (up_v * jax.nn.sigmoid(up_v) * up_g).astype(jnp.bfloat16)
    part = lax.dot_general(hid, s2_ref[...], dn, preferred_element_type=jnp.float32)

    @pl.when(j == 0)
    def _():
        acc_ref[...] = part

    @pl.when(j > 0)
    def _():
        acc_ref[...] += part

    @pl.when(j == nj - 1)
    def _():
        out_ref[...] = acc_ref[...] + b2_ref[...]


def _shared_mlp(x_flat, sh_fc1_w, sh_fc1_b, sh_fc2_w, sh_fc2_b, tile):
    t, h = x_flat.shape
    nt = t // tile
    sh_inter = sh_fc2_w.shape[1]
    sp = -(-sh_inter // 768) * 768
    sib = 768
    nj = sp // sib
    pad = sp - sh_inter
    sv = jnp.pad(sh_fc1_w[:sh_inter, :], ((0, pad), (0, 0))).astype(jnp.bfloat16)
    sg = jnp.pad(sh_fc1_w[sh_inter:, :], ((0, pad), (0, 0))).astype(jnp.bfloat16)
    s2b = jnp.pad(sh_fc2_w, ((0, 0), (0, pad))).astype(jnp.bfloat16)
    bv = jnp.pad(sh_fc1_b[:sh_inter], (0, pad)).reshape(1, sp)
    bg = jnp.pad(sh_fc1_b[sh_inter:], (0, pad)).reshape(1, sp)
    b2 = sh_fc2_b.reshape(1, h)
    return pl.pallas_call(
        _shared_body,
        grid=(nt, nj),
        in_specs=[
            pl.BlockSpec((tile, h), lambda i, j: (i, 0)),
            pl.BlockSpec((sib, h), lambda i, j: (j, 0)),
            pl.BlockSpec((sib, h), lambda i, j: (j, 0)),
            pl.BlockSpec((h, sib), lambda i, j: (0, j)),
            pl.BlockSpec((1, sib), lambda i, j: (0, j)),
            pl.BlockSpec((1, sib), lambda i, j: (0, j)),
            pl.BlockSpec((1, h), lambda i, j: (0, 0)),
        ],
        out_specs=pl.BlockSpec((tile, h), lambda i, j: (i, 0)),
        out_shape=jax.ShapeDtypeStruct((t, h), jnp.float32),
        scratch_shapes=[pltpu.VMEM((tile, h), jnp.float32)],
    )(x_flat, sv, sg, s2b, bv, bg, b2)


def _combine_body(sh_ref, g0_ref, g1_ref, out_ref):
    out_ref[...] = (sh_ref[...] + g0_ref[...].astype(jnp.float32)
                    + g1_ref[...].astype(jnp.float32))


def _combine(shared_out, gcat, tile):
    t, h = shared_out.shape
    nt = t // tile
    return pl.pallas_call(
        _combine_body,
        grid=(nt,),
        in_specs=[
            pl.BlockSpec((tile, h), lambda i: (i, 0)),
            pl.BlockSpec((tile, h), lambda i: (i, 0)),
            pl.BlockSpec((tile, h), lambda i: (i + nt, 0)),
        ],
        out_specs=pl.BlockSpec((tile, h), lambda i: (i, 0)),
        out_shape=jax.ShapeDtypeStruct((t, h), jnp.float32),
    )(shared_out, gcat, gcat)


def kernel(x, gate_weight, gate_bias, fc1_w, fc1_b, fc2_w, fc2_b,
           sh_fc1_w, sh_fc1_b, sh_fc2_w, sh_fc2_b):
    b, s, h = x.shape
    t = b * s
    e = gate_weight.shape[0]
    x_flat = x.reshape(t, h)
    tile = min(256, t)
    p = 2 * t
    max_tiles = p // tile + e
    r = max_tiles * tile

    i1, i2, w1, w2 = _route(x_flat, gate_weight, gate_bias)

    e_all = jnp.concatenate([i1, i2])
    w_all = jnp.concatenate([w1, w2])
    perm = jnp.argsort(e_all, stable=True)
    e_sorted = e_all[perm]
    tok_sorted = (perm % t).astype(jnp.int32)
    counts = jnp.bincount(e_all, length=e)
    tiles_per_e = (counts + tile - 1) // tile
    seg_start = (jnp.concatenate([jnp.zeros((1,), jnp.int32),
                                  jnp.cumsum(tiles_per_e)[:-1].astype(jnp.int32)]) * tile)
    orig_start = jnp.concatenate([jnp.zeros((1,), jnp.int32),
                                  jnp.cumsum(counts)[:-1].astype(jnp.int32)])
    rank = jnp.arange(p, dtype=jnp.int32) - orig_start[e_sorted]
    dst = seg_start[e_sorted] + rank
    row_tok = jnp.zeros((r,), jnp.int32).at[dst].set(tok_sorted)
    row_w = jnp.zeros((r,), jnp.float32).at[dst].set(w_all[perm])
    inv = jnp.zeros((p,), jnp.int32).at[perm].set(dst)
    num_tiles_used = jnp.sum(tiles_per_e).astype(jnp.int32)
    tile_expert = jnp.repeat(jnp.arange(e, dtype=jnp.int32), tiles_per_e,
                             total_repeat_length=max_tiles)
    tile_expert = jnp.clip(tile_expert, 0, e - 1).astype(jnp.int32)
    tile_valid = (jnp.arange(max_tiles, dtype=jnp.int32) < num_tiles_used).astype(jnp.int32)

    shared_out = _shared_mlp(x_flat, sh_fc1_w, sh_fc1_b, sh_fc2_w, sh_fc2_b, tile)

    x_sorted = _sc_gather(x_flat, row_tok)

    y_sorted = _grouped_mlp(x_sorted, row_w, tile_expert, tile_valid,
                            fc1_w, fc1_b, fc2_w, fc2_b, tile)

    gcat = _sc_gather(y_sorted, inv)

    out = _combine(shared_out, gcat, tile)
    return out.reshape(b, s, h)


# speedup vs baseline: 1.9396x; 1.0792x over previous
import functools

import jax
import jax.numpy as jnp
from jax import lax
from jax.experimental import pallas as pl
from jax.experimental.pallas import tpu as pltpu
from jax.experimental.pallas import tpu_sc as plsc

_SCALING = 2.5
_LANES = 128


def _sc_gather(table, idx):
    n, = idx.shape
    _, d = table.shape
    info = plsc.get_sparse_core_info()
    nc, ns = info.num_cores, info.num_subcores
    nw = nc * ns
    bpw = n // nw
    budget = 480 * 1024 // (2 * d * table.dtype.itemsize)
    chunk = next(c for c in (32, 24, 16, 8) if c <= budget and bpw % c == 0)
    assert n % nw == 0, (n, nw)
    nch = bpw // chunk
    mesh = plsc.VectorSubcoreMesh(core_axis_name="c", subcore_axis_name="s")

    @functools.partial(
        pl.kernel, mesh=mesh,
        out_type=jax.ShapeDtypeStruct((n, d), table.dtype),
        scratch_types=[
            pltpu.VMEM((bpw,), jnp.int32),
            pltpu.VMEM((2, chunk, d), table.dtype),
            pltpu.SemaphoreType.DMA,
            pltpu.SemaphoreType.DMA,
        ],
    )
    def k(table_hbm, idx_hbm, out_hbm, idx_v, rows_v, sem0, sem1):
        wid = lax.axis_index("s") * nc + lax.axis_index("c")
        base = wid * bpw
        pltpu.sync_copy(idx_hbm.at[pl.ds(base, bpw)], idx_v)
        sems = (sem0, sem1)
        cps = [None, None]
        cps[0] = pltpu.async_copy(
            table_hbm.at[idx_v.at[pl.ds(0, chunk)]], rows_v.at[0], sems[0])
        for c in range(nch):
            cur = c % 2
            nxt = (c + 1) % 2
            if c + 1 < nch:
                cps[nxt] = pltpu.async_copy(
                    table_hbm.at[idx_v.at[pl.ds((c + 1) * chunk, chunk)]],
                    rows_v.at[nxt], sems[nxt])
            cps[cur].wait()
            pltpu.sync_copy(rows_v.at[cur], out_hbm.at[pl.ds(base + c * chunk, chunk)])

    return k(table, idx)


def _router_body(x_ref, gwt_ref, bias_ref, idx_ref, w_ref):
    logits = jnp.dot(x_ref[...], gwt_ref[...], preferred_element_type=jnp.float32)
    s = jax.nn.sigmoid(logits) + bias_ref[...]
    lane = lax.broadcasted_iota(jnp.int32, s.shape, 1)
    big = jnp.int32(2 ** 30)
    v1 = jnp.max(s, axis=1, keepdims=True)
    i1 = jnp.min(jnp.where(s == v1, lane, big), axis=1, keepdims=True)
    s2 = jnp.where(lane == i1, jnp.float32(-1e30), s)
    v2 = jnp.max(s2, axis=1, keepdims=True)
    i2 = jnp.min(jnp.where(s2 == v2, lane, big), axis=1, keepdims=True)
    denom = v1 + v2 + 1e-20
    w1 = v1 / denom * _SCALING
    w2 = v2 / denom * _SCALING
    idx_ref[...] = jnp.where(lane == 0, i1, jnp.where(lane == 1, i2, 0)).astype(jnp.int32)
    w_ref[...] = jnp.where(lane == 0, w1, jnp.where(lane == 1, w2, 0.0))


def _route(x_flat, gate_weight, gate_bias):
    t, h = x_flat.shape
    e = gate_weight.shape[0]
    tb = min(256, t)
    gwt = jnp.zeros((h, _LANES), jnp.float32).at[:, :e].set(gate_weight.T)
    bias = jnp.full((1, _LANES), -1e30, jnp.float32).at[0, :e].set(gate_bias)
    idx, w = pl.pallas_call(
        _router_body,
        grid=(t // tb,),
        in_specs=[
            pl.BlockSpec((tb, h), lambda i: (i, 0)),
            pl.BlockSpec((h, _LANES), lambda i: (0, 0)),
            pl.BlockSpec((1, _LANES), lambda i: (0, 0)),
        ],
        out_specs=[
            pl.BlockSpec((tb, _LANES), lambda i: (i, 0)),
            pl.BlockSpec((tb, _LANES), lambda i: (i, 0)),
        ],
        out_shape=[
            jax.ShapeDtypeStruct((t, _LANES), jnp.int32),
            jax.ShapeDtypeStruct((t, _LANES), jnp.float32),
        ],
    )(x_flat, gwt, bias)
    return idx[:, 0], idx[:, 1], w[:, 0], w[:, 1]


def _mlp_body(te_ref, tv_ref, xs_ref, wv_ref, wg_ref, w2_ref, bv_ref, bg_ref,
              b2_ref, rw_ref, out_ref, acc_ref):
    i = pl.program_id(0)
    j = pl.program_id(1)
    nj = pl.num_programs(1)

    @pl.when(tv_ref[i] == 1)
    def _():
        x = xs_ref[...].astype(jnp.bfloat16)
        dn = (((1,), (1,)), ((), ()))
        up_v = lax.dot_general(x, wv_ref[0], dn, preferred_element_type=jnp.float32) + bv_ref[0]
        up_g = lax.dot_general(x, wg_ref[0], dn, preferred_element_type=jnp.float32) + bg_ref[0]
        hid = (up_v * jax.nn.sigmoid(up_v) * up_g).astype(jnp.bfloat16)
        part = lax.dot_general(hid, w2_ref[0], dn, preferred_element_type=jnp.float32)

        @pl.when(j == 0)
        def _():
            acc_ref[...] = part

        @pl.when(j > 0)
        def _():
            acc_ref[...] += part

        @pl.when(j == nj - 1)
        def _():
            out_ref[...] = (acc_ref[...] + b2_ref[0]) * rw_ref[:, :1]


def _grouped_mlp(x_sorted, row_w, tile_expert, tile_valid, fc1_w, fc1_b, fc2_w, fc2_b,
                 tile):
    r, h = x_sorted.shape
    e, two_i, _ = fc1_w.shape
    inter = two_i // 2
    ip = -(-inter // 512) * 512
    ib = 768 if ip % 768 == 0 else (512 if ip % 512 == 0 else ip)
    nj = ip // ib
    pad = ip - inter
    max_tiles = r // tile
    wv = jnp.pad(fc1_w[:, :inter, :], ((0, 0), (0, pad), (0, 0))).astype(jnp.bfloat16)
    wg = jnp.pad(fc1_w[:, inter:, :], ((0, 0), (0, pad), (0, 0))).astype(jnp.bfloat16)
    w2b = jnp.pad(fc2_w, ((0, 0), (0, 0), (0, pad))).astype(jnp.bfloat16)
    bv = jnp.pad(fc1_b[:, :inter], ((0, 0), (0, pad))).reshape(e, 1, ip)
    bg = jnp.pad(fc1_b[:, inter:], ((0, 0), (0, pad))).reshape(e, 1, ip)
    b2 = fc2_b.reshape(e, 1, h)
    rw2d = jnp.broadcast_to(row_w[:, None], (r, _LANES))

    grid_spec = pltpu.PrefetchScalarGridSpec(
        num_scalar_prefetch=2,
        grid=(max_tiles, nj),
        in_specs=[
            pl.BlockSpec((tile, h), lambda i, j, te, tv: (i, 0)),
            pl.BlockSpec((1, ib, h), lambda i, j, te, tv: (te[i], j, 0)),
            pl.BlockSpec((1, ib, h), lambda i, j, te, tv: (te[i], j, 0)),
            pl.BlockSpec((1, h, ib), lambda i, j, te, tv: (te[i], 0, j)),
            pl.BlockSpec((1, 1, ib), lambda i, j, te, tv: (te[i], 0, j)),
            pl.BlockSpec((1, 1, ib), lambda i, j, te, tv: (te[i], 0, j)),
            pl.BlockSpec((1, 1, h), lambda i, j, te, tv: (te[i], 0, 0)),
            pl.BlockSpec((tile, _LANES), lambda i, j, te, tv: (i, 0)),
        ],
        out_specs=pl.BlockSpec((tile, h), lambda i, j, te, tv: (i, 0)),
        scratch_shapes=[pltpu.VMEM((tile, h), jnp.float32)],
    )
    return pl.pallas_call(
        _mlp_body,
        grid_spec=grid_spec,
        out_shape=jax.ShapeDtypeStruct((r, h), jnp.float32),
    )(tile_expert, tile_valid, x_sorted, wv, wg, w2b, bv, bg, b2, rw2d)


def _shared_body(x_ref, sv_ref, sg_ref, s2_ref, bv_ref, bg_ref, b2_ref,
                 out_ref, acc_ref):
    j = pl.program_id(1)
    nj = pl.num_programs(1)
    x = x_ref[...].astype(jnp.bfloat16)
    dn = (((1,), (1,)), ((), ()))
    up_v = lax.dot_general(x, sv_ref[...], dn, preferred_element_type=jnp.float32) + bv_ref[...]
    up_g = lax.dot_general(x, sg_ref[...], dn, preferred_element_type=jnp.float32) + bg_ref[...]
    hid = (up_v * jax.nn.sigmoid(up_v) * up_g).astype(jnp.bfloat16)
    part = lax.dot_general(hid, s2_ref[...], dn, preferred_element_type=jnp.float32)

    @pl.when(j == 0)
    def _():
        acc_ref[...] = part

    @pl.when(j > 0)
    def _():
        acc_ref[...] += part

    @pl.when(j == nj - 1)
    def _():
        out_ref[...] = acc_ref[...] + b2_ref[...]


def _shared_mlp(x_flat, sh_fc1_w, sh_fc1_b, sh_fc2_w, sh_fc2_b, tile):
    t, h = x_flat.shape
    nt = t // tile
    sh_inter = sh_fc2_w.shape[1]
    sp = -(-sh_inter // 768) * 768
    sib = 768
    nj = sp // sib
    pad = sp - sh_inter
    sv = jnp.pad(sh_fc1_w[:sh_inter, :], ((0, pad), (0, 0))).astype(jnp.bfloat16)
    sg = jnp.pad(sh_fc1_w[sh_inter:, :], ((0, pad), (0, 0))).astype(jnp.bfloat16)
    s2b = jnp.pad(sh_fc2_w, ((0, 0), (0, pad))).astype(jnp.bfloat16)
    bv = jnp.pad(sh_fc1_b[:sh_inter], (0, pad)).reshape(1, sp)
    bg = jnp.pad(sh_fc1_b[sh_inter:], (0, pad)).reshape(1, sp)
    b2 = sh_fc2_b.reshape(1, h)
    return pl.pallas_call(
        _shared_body,
        grid=(nt, nj),
        in_specs=[
            pl.BlockSpec((tile, h), lambda i, j: (i, 0)),
            pl.BlockSpec((sib, h), lambda i, j: (j, 0)),
            pl.BlockSpec((sib, h), lambda i, j: (j, 0)),
            pl.BlockSpec((h, sib), lambda i, j: (0, j)),
            pl.BlockSpec((1, sib), lambda i, j: (0, j)),
            pl.BlockSpec((1, sib), lambda i, j: (0, j)),
            pl.BlockSpec((1, h), lambda i, j: (0, 0)),
        ],
        out_specs=pl.BlockSpec((tile, h), lambda i, j: (i, 0)),
        out_shape=jax.ShapeDtypeStruct((t, h), jnp.float32),
        scratch_shapes=[pltpu.VMEM((tile, h), jnp.float32)],
    )(x_flat, sv, sg, s2b, bv, bg, b2)


def _combine_body(sh_ref, g0_ref, g1_ref, out_ref):
    out_ref[...] = (sh_ref[...] + g0_ref[...].astype(jnp.float32)
                    + g1_ref[...].astype(jnp.float32))


def _combine(shared_out, gcat, tile):
    t, h = shared_out.shape
    nt = t // tile
    return pl.pallas_call(
        _combine_body,
        grid=(nt,),
        in_specs=[
            pl.BlockSpec((tile, h), lambda i: (i, 0)),
            pl.BlockSpec((tile, h), lambda i: (i, 0)),
            pl.BlockSpec((tile, h), lambda i: (i + nt, 0)),
        ],
        out_specs=pl.BlockSpec((tile, h), lambda i: (i, 0)),
        out_shape=jax.ShapeDtypeStruct((t, h), jnp.float32),
    )(shared_out, gcat, gcat)


def kernel(x, gate_weight, gate_bias, fc1_w, fc1_b, fc2_w, fc2_b,
           sh_fc1_w, sh_fc1_b, sh_fc2_w, sh_fc2_b):
    b, s, h = x.shape
    t = b * s
    e = gate_weight.shape[0]
    x_flat = x.reshape(t, h)
    tile = min(256, t)
    p = 2 * t
    max_tiles = p // tile + e
    r = max_tiles * tile

    i1, i2, w1, w2 = _route(x_flat, gate_weight, gate_bias)

    e_all = jnp.concatenate([i1, i2])
    w_all = jnp.concatenate([w1, w2])
    onehot = (e_all[:, None] == jnp.arange(e, dtype=jnp.int32)[None, :]).astype(jnp.int32)
    cum = jnp.cumsum(onehot, axis=0)
    counts = cum[-1]
    tiles_per_e = (counts + tile - 1) // tile
    seg_start = (jnp.concatenate([jnp.zeros((1,), jnp.int32),
                                  jnp.cumsum(tiles_per_e)[:-1].astype(jnp.int32)]) * tile)
    dst = jnp.sum(onehot * (seg_start[None, :] + cum - 1), axis=1).astype(jnp.int32)
    tok_all = (jnp.arange(p, dtype=jnp.int32) % t)
    row_tok = jnp.zeros((r,), jnp.int32).at[dst].set(tok_all)
    row_w = jnp.zeros((r,), jnp.float32).at[dst].set(w_all)
    inv = dst
    num_tiles_used = jnp.sum(tiles_per_e).astype(jnp.int32)
    tile_expert = jnp.repeat(jnp.arange(e, dtype=jnp.int32), tiles_per_e,
                             total_repeat_length=max_tiles)
    tile_expert = jnp.clip(tile_expert, 0, e - 1).astype(jnp.int32)
    tile_valid = (jnp.arange(max_tiles, dtype=jnp.int32) < num_tiles_used).astype(jnp.int32)

    shared_out = _shared_mlp(x_flat, sh_fc1_w, sh_fc1_b, sh_fc2_w, sh_fc2_b, tile)

    x_sorted = _sc_gather(x_flat, row_tok)

    y_sorted = _grouped_mlp(x_sorted, row_w, tile_expert, tile_valid,
                            fc1_w, fc1_b, fc2_w, fc2_b, tile)

    gcat = _sc_gather(y_sorted, inv)

    out = _combine(shared_out, gcat, tile)
    return out.reshape(b, s, h)


# speedup vs baseline: 2.1117x; 1.0888x over previous
import functools

import jax
import jax.numpy as jnp
from jax import lax
from jax.experimental import pallas as pl
from jax.experimental.pallas import tpu as pltpu
from jax.experimental.pallas import tpu_sc as plsc

_SCALING = 2.5
_LANES = 128


def _sc_gather(table, idx):
    n, = idx.shape
    _, d = table.shape
    info = plsc.get_sparse_core_info()
    nc, ns = info.num_cores, info.num_subcores
    nw = nc * ns
    bpw = n // nw
    budget = 480 * 1024 // (2 * d * table.dtype.itemsize)
    chunk = next(c for c in (32, 24, 16, 8) if c <= budget and bpw % c == 0)
    assert n % nw == 0, (n, nw)
    nch = bpw // chunk
    mesh = plsc.VectorSubcoreMesh(core_axis_name="c", subcore_axis_name="s")

    @functools.partial(
        pl.kernel, mesh=mesh,
        out_type=jax.ShapeDtypeStruct((n, d), table.dtype),
        scratch_types=[
            pltpu.VMEM((bpw,), jnp.int32),
            pltpu.VMEM((2, chunk, d), table.dtype),
            pltpu.SemaphoreType.DMA,
            pltpu.SemaphoreType.DMA,
        ],
    )
    def k(table_hbm, idx_hbm, out_hbm, idx_v, rows_v, sem0, sem1):
        wid = lax.axis_index("s") * nc + lax.axis_index("c")
        base = wid * bpw
        pltpu.sync_copy(idx_hbm.at[pl.ds(base, bpw)], idx_v)
        sems = (sem0, sem1)
        cps = [None, None]
        cps[0] = pltpu.async_copy(
            table_hbm.at[idx_v.at[pl.ds(0, chunk)]], rows_v.at[0], sems[0])
        for c in range(nch):
            cur = c % 2
            nxt = (c + 1) % 2
            if c + 1 < nch:
                cps[nxt] = pltpu.async_copy(
                    table_hbm.at[idx_v.at[pl.ds((c + 1) * chunk, chunk)]],
                    rows_v.at[nxt], sems[nxt])
            cps[cur].wait()
            pltpu.sync_copy(rows_v.at[cur], out_hbm.at[pl.ds(base + c * chunk, chunk)])

    return k(table, idx)


def _router_body(x_ref, gwt_ref, bias_ref, idx_ref, w_ref):
    logits = jnp.dot(x_ref[...], gwt_ref[...], preferred_element_type=jnp.float32)
    s = jax.nn.sigmoid(logits) + bias_ref[...]
    lane = lax.broadcasted_iota(jnp.int32, s.shape, 1)
    big = jnp.int32(2 ** 30)
    v1 = jnp.max(s, axis=1, keepdims=True)
    i1 = jnp.min(jnp.where(s == v1, lane, big), axis=1, keepdims=True)
    s2 = jnp.where(lane == i1, jnp.float32(-1e30), s)
    v2 = jnp.max(s2, axis=1, keepdims=True)
    i2 = jnp.min(jnp.where(s2 == v2, lane, big), axis=1, keepdims=True)
    denom = v1 + v2 + 1e-20
    w1 = v1 / denom * _SCALING
    w2 = v2 / denom * _SCALING
    idx_ref[...] = jnp.where(lane == 0, i1, jnp.where(lane == 1, i2, 0)).astype(jnp.int32)
    w_ref[...] = jnp.where(lane == 0, w1, jnp.where(lane == 1, w2, 0.0))


def _route(x_flat, gate_weight, gate_bias):
    t, h = x_flat.shape
    e = gate_weight.shape[0]
    tb = min(256, t)
    gwt = jnp.zeros((h, _LANES), jnp.float32).at[:, :e].set(gate_weight.T)
    bias = jnp.full((1, _LANES), -1e30, jnp.float32).at[0, :e].set(gate_bias)
    idx, w = pl.pallas_call(
        _router_body,
        grid=(t // tb,),
        in_specs=[
            pl.BlockSpec((tb, h), lambda i: (i, 0)),
            pl.BlockSpec((h, _LANES), lambda i: (0, 0)),
            pl.BlockSpec((1, _LANES), lambda i: (0, 0)),
        ],
        out_specs=[
            pl.BlockSpec((tb, _LANES), lambda i: (i, 0)),
            pl.BlockSpec((tb, _LANES), lambda i: (i, 0)),
        ],
        out_shape=[
            jax.ShapeDtypeStruct((t, _LANES), jnp.int32),
            jax.ShapeDtypeStruct((t, _LANES), jnp.float32),
        ],
    )(x_flat, gwt, bias)
    return idx[:, 0], idx[:, 1], w[:, 0], w[:, 1]


def _mlp_body(te_ref, tv_ref, xs_ref, wv_ref, wg_ref, w2_ref, bv_ref, bg_ref,
              b2_ref, rw_ref, out_ref, acc_ref):
    i = pl.program_id(0)
    j = pl.program_id(1)
    nj = pl.num_programs(1)

    @pl.when(tv_ref[i] == 1)
    def _():
        x = xs_ref[...].astype(jnp.bfloat16)
        dn = (((1,), (1,)), ((), ()))
        up_v = lax.dot_general(x, wv_ref[0], dn, preferred_element_type=jnp.float32) + bv_ref[0]
        up_g = lax.dot_general(x, wg_ref[0], dn, preferred_element_type=jnp.float32) + bg_ref[0]
        hid = (up_v * jax.nn.sigmoid(up_v) * up_g).astype(jnp.bfloat16)
        part = lax.dot_general(hid, w2_ref[0], dn, preferred_element_type=jnp.float32)

        @pl.when(j == 0)
        def _():
            acc_ref[...] = part

        @pl.when(j > 0)
        def _():
            acc_ref[...] += part

        @pl.when(j == nj - 1)
        def _():
            out_ref[...] = (acc_ref[...] + b2_ref[0]) * rw_ref[:, :1]


def _grouped_mlp(x_sorted, row_w, tile_expert, tile_valid, fc1_w, fc1_b, fc2_w, fc2_b,
                 tile):
    r = x_sorted.shape[0]
    e, two_i, h = fc1_w.shape
    inter = two_i // 2
    ip = -(-inter // 512) * 512
    ib = ip
    nj = ip // ib
    pad = ip - inter
    max_tiles = r // tile
    wv = jnp.pad(fc1_w[:, :inter, :], ((0, 0), (0, pad), (0, 0))).astype(jnp.bfloat16)
    wg = jnp.pad(fc1_w[:, inter:, :], ((0, 0), (0, pad), (0, 0))).astype(jnp.bfloat16)
    w2b = jnp.pad(fc2_w, ((0, 0), (0, 0), (0, pad))).astype(jnp.bfloat16)
    bv = jnp.pad(fc1_b[:, :inter], ((0, 0), (0, pad))).reshape(e, 1, ip)
    bg = jnp.pad(fc1_b[:, inter:], ((0, 0), (0, pad))).reshape(e, 1, ip)
    b2 = fc2_b.reshape(e, 1, h)
    rw2d = jnp.broadcast_to(row_w[:, None], (r, _LANES))

    grid_spec = pltpu.PrefetchScalarGridSpec(
        num_scalar_prefetch=2,
        grid=(max_tiles, nj),
        in_specs=[
            pl.BlockSpec((tile, h), lambda i, j, te, tv: (i, 0)),
            pl.BlockSpec((1, ib, h), lambda i, j, te, tv: (te[i], j, 0)),
            pl.BlockSpec((1, ib, h), lambda i, j, te, tv: (te[i], j, 0)),
            pl.BlockSpec((1, h, ib), lambda i, j, te, tv: (te[i], 0, j)),
            pl.BlockSpec((1, 1, ib), lambda i, j, te, tv: (te[i], 0, j)),
            pl.BlockSpec((1, 1, ib), lambda i, j, te, tv: (te[i], 0, j)),
            pl.BlockSpec((1, 1, h), lambda i, j, te, tv: (te[i], 0, 0)),
            pl.BlockSpec((tile, _LANES), lambda i, j, te, tv: (i, 0)),
        ],
        out_specs=pl.BlockSpec((tile, h), lambda i, j, te, tv: (i, 0)),
        scratch_shapes=[pltpu.VMEM((tile, h), jnp.float32)],
    )
    return pl.pallas_call(
        _mlp_body,
        grid_spec=grid_spec,
        out_shape=jax.ShapeDtypeStruct((r, h), jnp.float32),
    )(tile_expert, tile_valid, x_sorted, wv, wg, w2b, bv, bg, b2, rw2d)


def _shared_body(x_ref, sv_ref, sg_ref, s2_ref, bv_ref, bg_ref, b2_ref,
                 out_ref, acc_ref):
    j = pl.program_id(1)
    nj = pl.num_programs(1)
    x = x_ref[...].astype(jnp.bfloat16)
    dn = (((1,), (1,)), ((), ()))
    up_v = lax.dot_general(x, sv_ref[...], dn, preferred_element_type=jnp.float32) + bv_ref[...]
    up_g = lax.dot_general(x, sg_ref[...], dn, preferred_element_type=jnp.float32) + bg_ref[...]
    hid = (up_v * jax.nn.sigmoid(up_v) * up_g).astype(jnp.bfloat16)
    part = lax.dot_general(hid, s2_ref[...], dn, preferred_element_type=jnp.float32)

    @pl.when(j == 0)
    def _():
        acc_ref[...] = part

    @pl.when(j > 0)
    def _():
        acc_ref[...] += part

    @pl.when(j == nj - 1)
    def _():
        out_ref[...] = acc_ref[...] + b2_ref[...]


def _shared_mlp(x_flat, sh_fc1_w, sh_fc1_b, sh_fc2_w, sh_fc2_b, tile):
    t, h = x_flat.shape
    nt = t // tile
    sh_inter = sh_fc2_w.shape[1]
    sp = -(-sh_inter // 768) * 768
    sib = 768
    nj = sp // sib
    pad = sp - sh_inter
    sv = jnp.pad(sh_fc1_w[:sh_inter, :], ((0, pad), (0, 0))).astype(jnp.bfloat16)
    sg = jnp.pad(sh_fc1_w[sh_inter:, :], ((0, pad), (0, 0))).astype(jnp.bfloat16)
    s2b = jnp.pad(sh_fc2_w, ((0, 0), (0, pad))).astype(jnp.bfloat16)
    bv = jnp.pad(sh_fc1_b[:sh_inter], (0, pad)).reshape(1, sp)
    bg = jnp.pad(sh_fc1_b[sh_inter:], (0, pad)).reshape(1, sp)
    b2 = sh_fc2_b.reshape(1, h)
    return pl.pallas_call(
        _shared_body,
        grid=(nt, nj),
        in_specs=[
            pl.BlockSpec((tile, h), lambda i, j: (i, 0)),
            pl.BlockSpec((sib, h), lambda i, j: (j, 0)),
            pl.BlockSpec((sib, h), lambda i, j: (j, 0)),
            pl.BlockSpec((h, sib), lambda i, j: (0, j)),
            pl.BlockSpec((1, sib), lambda i, j: (0, j)),
            pl.BlockSpec((1, sib), lambda i, j: (0, j)),
            pl.BlockSpec((1, h), lambda i, j: (0, 0)),
        ],
        out_specs=pl.BlockSpec((tile, h), lambda i, j: (i, 0)),
        out_shape=jax.ShapeDtypeStruct((t, h), jnp.float32),
        scratch_shapes=[pltpu.VMEM((tile, h), jnp.float32)],
    )(x_flat, sv, sg, s2b, bv, bg, b2)


def _combine_body(sh_ref, g0_ref, g1_ref, out_ref):
    out_ref[...] = (sh_ref[...] + g0_ref[...] + g1_ref[...])


def _combine(shared_out, gcat, tile):
    t, h = shared_out.shape
    nt = t // tile
    return pl.pallas_call(
        _combine_body,
        grid=(nt,),
        in_specs=[
            pl.BlockSpec((tile, h), lambda i: (i, 0)),
            pl.BlockSpec((tile, h), lambda i: (i, 0)),
            pl.BlockSpec((tile, h), lambda i: (i + nt, 0)),
        ],
        out_specs=pl.BlockSpec((tile, h), lambda i: (i, 0)),
        out_shape=jax.ShapeDtypeStruct((t, h), jnp.float32),
    )(shared_out, gcat, gcat)


def kernel(x, gate_weight, gate_bias, fc1_w, fc1_b, fc2_w, fc2_b,
           sh_fc1_w, sh_fc1_b, sh_fc2_w, sh_fc2_b):
    b, s, h = x.shape
    t = b * s
    e = gate_weight.shape[0]
    x_flat = x.reshape(t, h)
    tile = min(256, t)
    p = 2 * t
    max_tiles = p // tile + e
    r = max_tiles * tile

    i1, i2, w1, w2 = _route(x_flat, gate_weight, gate_bias)

    e_all = jnp.concatenate([i1, i2])
    w_all = jnp.concatenate([w1, w2])
    onehot = (e_all[:, None] == jnp.arange(e, dtype=jnp.int32)[None, :]).astype(jnp.int32)
    cum = jnp.cumsum(onehot, axis=0)
    counts = cum[-1]
    tiles_per_e = (counts + tile - 1) // tile
    seg_start = (jnp.concatenate([jnp.zeros((1,), jnp.int32),
                                  jnp.cumsum(tiles_per_e)[:-1].astype(jnp.int32)]) * tile)
    dst = jnp.sum(onehot * (seg_start[None, :] + cum - 1), axis=1).astype(jnp.int32)
    tok_all = (jnp.arange(p, dtype=jnp.int32) % t)
    row_tok = jnp.zeros((r,), jnp.int32).at[dst].set(tok_all)
    row_w = jnp.zeros((r,), jnp.float32).at[dst].set(w_all)
    inv = dst
    num_tiles_used = jnp.sum(tiles_per_e).astype(jnp.int32)
    tile_expert = jnp.repeat(jnp.arange(e, dtype=jnp.int32), tiles_per_e,
                             total_repeat_length=max_tiles)
    tile_expert = jnp.clip(tile_expert, 0, e - 1).astype(jnp.int32)
    tile_valid = (jnp.arange(max_tiles, dtype=jnp.int32) < num_tiles_used).astype(jnp.int32)

    shared_out = _shared_mlp(x_flat, sh_fc1_w, sh_fc1_b, sh_fc2_w, sh_fc2_b,
                             min(512, t))

    x_sorted = _sc_gather(x_flat, row_tok)

    y_sorted = _grouped_mlp(x_sorted, row_w, tile_expert, tile_valid,
                            fc1_w, fc1_b, fc2_w, fc2_b, tile)

    gcat = _sc_gather(y_sorted, inv)

    out = _combine(shared_out, gcat, tile)
    return out.reshape(b, s, h)
